# baseline (speedup 1.0000x reference)
import functools

import jax
import jax.numpy as jnp
from jax import lax
from jax.experimental import pallas as pl
from jax.experimental.pallas import tpu as pltpu
from jax.experimental.pallas import tpu_sc as plsc


def _iota16():
    return lax.broadcasted_iota(jnp.int32, (16,), 0)


@functools.lru_cache(maxsize=None)
def _make_kernels(seq: int, batch: int, vocab: int, dim: int):
    info = plsc.get_sparse_core_info()
    num_cores, num_subcores = info.num_cores, info.num_subcores
    num_workers = num_cores * num_subcores
    lanes = 128

    n_full_tcols = vocab // lanes
    tail = vocab - n_full_tcols * lanes
    rows_per_tcol = dim
    scratch_rows = (n_full_tcols + (1 if tail else 0)) * rows_per_tcol

    mesh = plsc.VectorSubcoreMesh(core_axis_name="c", subcore_axis_name="s")

    steps = (n_full_tcols + num_workers - 1) // num_workers
    half = (steps + 1) // 2

    @functools.partial(
        pl.kernel,
        mesh=mesh,
        compiler_params=pltpu.CompilerParams(needs_layout_passes=False),
        out_type=jax.ShapeDtypeStruct((scratch_rows, lanes), jnp.float32),
        scratch_types=[
            pltpu.VMEM((dim, lanes), jnp.float32),
            pltpu.VMEM((dim, lanes), jnp.float32),
            pltpu.VMEM((dim, lanes), jnp.float32),
            pltpu.VMEM((dim, lanes), jnp.float32),
            pltpu.SemaphoreType.DMA,
            pltpu.SemaphoreType.DMA,
            pltpu.SemaphoreType.DMA,
            pltpu.SemaphoreType.DMA,
        ],
    )
    def relayout_kernel(tab_hbm, tail_hbm, scr_hbm, vin0, vin1, vout0, vout1,
                        isem0, isem1, osem0, osem1):
        wid = lax.axis_index("s") * num_cores + lax.axis_index("c")
        vins = (vin0, vin1)
        vouts = (vout0, vout1)
        isems = (isem0, isem1)
        osems = (osem0, osem1)
        iota = _iota16()

        def col_of(k2):
            return wid + k2 * num_workers

        def fire_in(k2, p):
            c = col_of(k2)

            @pl.when(c < n_full_tcols)
            def _():
                off = pl.multiple_of(c * lanes, lanes)
                pltpu.async_copy(tab_hbm.at[:, pl.ds(off, lanes)],
                                 vins[p], isems[p])

        def transpose_tile(p, nb):
            vin, vout = vins[p], vouts[p]
            for b0 in range(nb):
                row = b0 // 4
                colb = 32 * (b0 % 4)
                cvec = jnp.full((16,), b0, jnp.int32)
                lo = plsc.load_gather(vin, [iota, cvec])
                hi = plsc.load_gather(vin, [iota + 16, cvec])
                vout[row, pl.ds(colb, 16)] = lo
                vout[row, pl.ds(colb + 16, 16)] = hi

        def fire_out(k2, p):
            c = col_of(k2)

            @pl.when(c < n_full_tcols)
            def _():
                off = pl.multiple_of(c * rows_per_tcol, 8)
                pltpu.async_copy(vouts[p],
                                 scr_hbm.at[pl.ds(off, rows_per_tcol), :],
                                 osems[p])

        def wait_in(k2, p):
            c = col_of(k2)

            @pl.when(c < n_full_tcols)
            def _():
                pltpu.make_async_copy(tab_hbm.at[:, pl.ds(0, lanes)],
                                      vins[p], isems[p]).wait()

        def wait_out(k2, p):
            c = col_of(k2)

            @pl.when(c < n_full_tcols)
            def _():
                pltpu.make_async_copy(vouts[p],
                                      scr_hbm.at[pl.ds(0, rows_per_tcol), :],
                                      osems[p]).wait()

        fire_in(0, 0)

        def body(k, carry):
            for h in range(2):
                k2 = 2 * k + h
                p = h
                fire_in(k2 + 1, 1 - p)
                wait_in(k2, p)

                @pl.when(col_of(k2) < n_full_tcols)
                def _():
                    transpose_tile(p, lanes)

                @pl.when(k2 >= 2)
                def _():
                    wait_out(k2 - 2, p)

                fire_out(k2, p)
            return carry

        lax.fori_loop(0, half, body, 0)
        wait_out(2 * half - 2, 0)
        wait_out(2 * half - 1, 1)

        if tail:
            tail_rows = tail * dim // lanes

            @pl.when(wid == num_workers - 1)
            def _():
                pltpu.sync_copy(tail_hbm, vin0.at[pl.ds(0, tail_rows), :])
                pltpu.sync_copy(
                    vin0.at[pl.ds(0, tail_rows), :],
                    scr_hbm.at[pl.ds(n_full_tcols * rows_per_tcol,
                                     tail_rows), :])

    @functools.partial(
        pl.kernel,
        mesh=mesh,
        compiler_params=pltpu.CompilerParams(needs_layout_passes=False),
        out_type=jax.ShapeDtypeStruct((seq, dim, batch), jnp.float32),
        scratch_types=[
            pltpu.VMEM((lanes,), jnp.int32),
            pltpu.VMEM((lanes,), jnp.int32),
            pltpu.VMEM((lanes,), jnp.int32),
            pltpu.VMEM((lanes,), jnp.int32),
            pltpu.VMEM((lanes,), jnp.int32),
            pltpu.VMEM((lanes,), jnp.int32),
            pltpu.VMEM((lanes, lanes), jnp.float32),
            pltpu.VMEM((lanes, lanes), jnp.float32),
            pltpu.VMEM((dim, lanes), jnp.float32),
            pltpu.VMEM((dim, lanes), jnp.float32),
            pltpu.SemaphoreType.DMA,
            pltpu.SemaphoreType.DMA,
            pltpu.SemaphoreType.DMA,
            pltpu.SemaphoreType.DMA,
            pltpu.SemaphoreType.DMA,
            pltpu.SemaphoreType.DMA,
        ],
    )
    def gather_kernel(idx_hbm, scr_hbm, out_hbm,
                      vidx0, vidx1, jref0, jref1, rbref0, rbref1,
                      g0, g1, vo0, vo1,
                      xsem0, xsem1, gsem0, gsem1, osem0, osem1):
        wid = lax.axis_index("s") * num_cores + lax.axis_index("c")
        boff = pl.multiple_of(wid * lanes, lanes)
        vidxs = (vidx0, vidx1)
        jrefs = (jref0, jref1)
        rbrefs = (rbref0, rbref1)
        gs = (g0, g1)
        vos = (vo0, vo1)
        xsems = (xsem0, xsem1)
        gsems = (gsem0, gsem1)
        osems = (osem0, osem1)
        iota = _iota16()

        def fire_idx(s, p):
            pltpu.async_copy(idx_hbm.at[s, pl.ds(boff, lanes)],
                             vidxs[p], xsems[p])

        def wait_idx(p):
            pltpu.make_async_copy(idx_hbm.at[0, pl.ds(boff, lanes)],
                                  vidxs[p], xsems[p]).wait()

        def fire_gather(p):
            for g in range(lanes // 16):
                iv = vidxs[p][pl.ds(16 * g, 16)]
                jrefs[p][pl.ds(16 * g, 16)] = lax.shift_right_logical(iv, 2)
                rbrefs[p][pl.ds(16 * g, 16)] = lax.shift_left(
                    lax.bitwise_and(iv, 3), 5)
            pltpu.async_copy(scr_hbm.at[jrefs[p]], gs[p], gsems[p])

        def wait_gather(p):
            pltpu.make_async_copy(scr_hbm.at[jrefs[p]], gs[p], gsems[p]).wait()

        def extract(p):
            G, vout = gs[p], vos[p]
            for g in range(lanes // 16):
                colbase = rbrefs[p][pl.ds(16 * g, 16)]
                rowidx = iota + (16 * g)
                for d in range(dim):
                    val = plsc.load_gather(G, [rowidx, colbase + d])
                    vout[d, pl.ds(16 * g, 16)] = val

        def fire_out(s, p):
            pltpu.async_copy(vos[p], out_hbm.at[s].at[:, pl.ds(boff, lanes)],
                             osems[p])

        def wait_out(p):
            pltpu.make_async_copy(vos[p],
                                  out_hbm.at[0].at[:, pl.ds(boff, lanes)],
                                  osems[p]).wait()

        fire_idx(0, 0)
        wait_idx(0)
        fire_gather(0)
        fire_idx(1, 1)

        def body(k, carry):
            for h in range(2):
                s = 2 * k + h
                p = h

                @pl.when(s + 1 < seq)
                def _():
                    wait_idx(1 - p)
                    fire_gather(1 - p)

                @pl.when(s + 2 < seq)
                def _():
                    fire_idx(s + 2, p)

                wait_gather(p)

                @pl.when(s >= 2)
                def _():
                    wait_out(p)

                extract(p)
                fire_out(s, p)
            return carry

        lax.fori_loop(0, seq // 2, body, 0)
        wait_out(0)
        wait_out(1)

    return relayout_kernel, gather_kernel


def kernel(indices, table):
    batch, seq = indices.shape
    vocab, dim = table.shape
    relayout_kernel, gather_kernel = _make_kernels(seq, batch, vocab, dim)
    idx_t = jnp.transpose(indices.astype(jnp.int32))
    table_t = jnp.transpose(table)
    lanes = 128
    tail = vocab % lanes
    if tail:
        tail_rows = table[vocab - tail:].reshape(tail * dim // lanes, lanes)
    else:
        tail_rows = jnp.zeros((8, lanes), jnp.float32)
    scratch = relayout_kernel(table_t, tail_rows)
    out_t = gather_kernel(idx_t, scratch)
    return jnp.transpose(out_t, (2, 0, 1))


# speedup vs baseline: 2.4715x; 2.4715x over previous
import functools

import jax
import jax.numpy as jnp
from jax import lax
from jax.experimental import pallas as pl
from jax.experimental.pallas import tpu as pltpu
from jax.experimental.pallas import tpu_sc as plsc


def _iota16():
    return lax.broadcasted_iota(jnp.int32, (16,), 0)


@functools.lru_cache(maxsize=None)
def _make_kernels(seq: int, batch: int, vocab: int, dim: int):
    info = plsc.get_sparse_core_info()
    num_cores, num_subcores = info.num_cores, info.num_subcores
    num_workers = num_cores * num_subcores
    lanes = 128

    n_full_tcols = vocab // lanes
    tail = vocab - n_full_tcols * lanes
    rows_per_tcol = dim
    scratch_rows = (n_full_tcols + (1 if tail else 0)) * rows_per_tcol

    mesh = plsc.VectorSubcoreMesh(core_axis_name="c", subcore_axis_name="s")

    steps = (n_full_tcols + num_workers - 1) // num_workers
    half = (steps + 1) // 2

    scratch_words = scratch_rows * lanes

    @functools.partial(
        pl.kernel,
        mesh=mesh,
        compiler_params=pltpu.CompilerParams(needs_layout_passes=False),
        out_type=jax.ShapeDtypeStruct((scratch_words,), jnp.float32),
        scratch_types=[
            pltpu.VMEM((dim, lanes), jnp.float32),
            pltpu.VMEM((dim, lanes), jnp.float32),
            pltpu.VMEM((dim * lanes,), jnp.float32),
            pltpu.VMEM((dim * lanes,), jnp.float32),
            pltpu.SemaphoreType.DMA,
            pltpu.SemaphoreType.DMA,
            pltpu.SemaphoreType.DMA,
            pltpu.SemaphoreType.DMA,
        ],
    )
    def relayout_kernel(tab_hbm, tail_hbm, scr_hbm, vin0, vin1, vout0, vout1,
                        isem0, isem1, osem0, osem1):
        wid = lax.axis_index("s") * num_cores + lax.axis_index("c")
        vins = (vin0, vin1)
        vouts = (vout0, vout1)
        isems = (isem0, isem1)
        osems = (osem0, osem1)
        iota = _iota16()
        blk_words = dim * lanes

        def col_of(k2):
            return wid + k2 * num_workers

        def fire_in(k2, p):
            c = col_of(k2)

            @pl.when(c < n_full_tcols)
            def _():
                off = pl.multiple_of(c * lanes, lanes)
                pltpu.async_copy(tab_hbm.at[:, pl.ds(off, lanes)],
                                 vins[p], isems[p])

        def transpose_tile(p):
            vin, vout = vins[p], vouts[p]
            rvecs = [iota + 16 * dblk for dblk in range(dim // 16)]

            def tbody(t, carry):
                pt = lax.bitwise_and(iota + t, 15)
                pt32i = lax.shift_left(pt, 5) + iota
                for dblk in range(dim // 16):
                    for bblk in range(lanes // 16):
                        b0 = 16 * bblk
                        src = plsc.load_gather(vin, [rvecs[dblk], pt + b0])
                        plsc.store_scatter(
                            vout, [pt32i + (b0 * 32 + 16 * dblk)], src)
                return carry

            lax.fori_loop(0, 16, tbody, 0)

        def fire_out(k2, p):
            c = col_of(k2)

            @pl.when(c < n_full_tcols)
            def _():
                off = pl.multiple_of(c * blk_words, 8)
                pltpu.async_copy(vouts[p],
                                 scr_hbm.at[pl.ds(off, blk_words)],
                                 osems[p])

        def wait_in(k2, p):
            c = col_of(k2)

            @pl.when(c < n_full_tcols)
            def _():
                pltpu.make_async_copy(tab_hbm.at[:, pl.ds(0, lanes)],
                                      vins[p], isems[p]).wait()

        def wait_out(k2, p):
            c = col_of(k2)

            @pl.when(c < n_full_tcols)
            def _():
                pltpu.make_async_copy(vouts[p],
                                      scr_hbm.at[pl.ds(0, blk_words)],
                                      osems[p]).wait()

        fire_in(0, 0)

        def body(k, carry):
            for h in range(2):
                k2 = 2 * k + h
                p = h
                fire_in(k2 + 1, 1 - p)
                wait_in(k2, p)

                @pl.when(col_of(k2) < n_full_tcols)
                def _():
                    transpose_tile(p)

                @pl.when(k2 >= 2)
                def _():
                    wait_out(k2 - 2, p)

                fire_out(k2, p)
            return carry

        lax.fori_loop(0, half, body, 0)
        wait_out(2 * half - 2, 0)
        wait_out(2 * half - 1, 1)

        if tail:
            tail_words = tail * dim

            @pl.when(wid == num_workers - 1)
            def _():
                pltpu.sync_copy(tail_hbm, vout0.at[pl.ds(0, tail_words)])
                pltpu.sync_copy(
                    vout0.at[pl.ds(0, tail_words)],
                    scr_hbm.at[pl.ds(n_full_tcols * blk_words, tail_words)])

    @functools.partial(
        pl.kernel,
        mesh=mesh,
        compiler_params=pltpu.CompilerParams(needs_layout_passes=False),
        out_type=jax.ShapeDtypeStruct((seq, dim, batch), jnp.float32),
        scratch_types=[
            pltpu.VMEM((lanes,), jnp.int32),
            pltpu.VMEM((lanes,), jnp.int32),
            pltpu.VMEM((lanes,), jnp.int32),
            pltpu.VMEM((lanes,), jnp.int32),
            pltpu.VMEM((lanes,), jnp.int32),
            pltpu.VMEM((lanes,), jnp.int32),
            pltpu.VMEM((lanes, lanes), jnp.float32),
            pltpu.VMEM((lanes, lanes), jnp.float32),
            pltpu.VMEM((dim, lanes), jnp.float32),
            pltpu.VMEM((dim, lanes), jnp.float32),
            pltpu.SemaphoreType.DMA,
            pltpu.SemaphoreType.DMA,
            pltpu.SemaphoreType.DMA,
            pltpu.SemaphoreType.DMA,
            pltpu.SemaphoreType.DMA,
            pltpu.SemaphoreType.DMA,
        ],
    )
    def gather_kernel(idx_hbm, scr_hbm, out_hbm,
                      vidx0, vidx1, jref0, jref1, rbref0, rbref1,
                      g0, g1, vo0, vo1,
                      xsem0, xsem1, gsem0, gsem1, osem0, osem1):
        wid = lax.axis_index("s") * num_cores + lax.axis_index("c")
        boff = pl.multiple_of(wid * lanes, lanes)
        vidxs = (vidx0, vidx1)
        jrefs = (jref0, jref1)
        rbrefs = (rbref0, rbref1)
        gs = (g0, g1)
        vos = (vo0, vo1)
        xsems = (xsem0, xsem1)
        gsems = (gsem0, gsem1)
        osems = (osem0, osem1)
        iota = _iota16()

        def fire_idx(s, p):
            pltpu.async_copy(idx_hbm.at[s, pl.ds(boff, lanes)],
                             vidxs[p], xsems[p])

        def wait_idx(p):
            pltpu.make_async_copy(idx_hbm.at[0, pl.ds(boff, lanes)],
                                  vidxs[p], xsems[p]).wait()

        def fire_gather(p):
            for g in range(lanes // 16):
                iv = vidxs[p][pl.ds(16 * g, 16)]
                jrefs[p][pl.ds(16 * g, 16)] = lax.shift_right_logical(iv, 2)
                rbrefs[p][pl.ds(16 * g, 16)] = lax.shift_left(
                    lax.bitwise_and(iv, 3), 5)
            pltpu.async_copy(scr_hbm.at[jrefs[p]], gs[p], gsems[p])

        def wait_gather(p):
            pltpu.make_async_copy(scr_hbm.at[jrefs[p]], gs[p], gsems[p]).wait()

        def extract(p):
            G, vout = gs[p], vos[p]
            colbases = [rbrefs[p][pl.ds(16 * g, 16)]
                        for g in range(lanes // 16)]
            rows = [iota + 16 * g for g in range(lanes // 16)]

            def tbody(t, carry):
                pt = lax.bitwise_and(iota + t, dim - 1)
                for g in range(lanes // 16):
                    val = plsc.load_gather(G, [rows[g], colbases[g] + pt])
                    plsc.store_scatter(vout, [pt, rows[g]], val)
                return carry

            lax.fori_loop(0, dim, tbody, 0)

        def fire_out(s, p):
            pltpu.async_copy(vos[p], out_hbm.at[s].at[:, pl.ds(boff, lanes)],
                             osems[p])

        def wait_out(p):
            pltpu.make_async_copy(vos[p],
                                  out_hbm.at[0].at[:, pl.ds(boff, lanes)],
                                  osems[p]).wait()

        fire_idx(0, 0)
        wait_idx(0)
        fire_gather(0)
        fire_idx(1, 1)

        def body(k, carry):
            for h in range(2):
                s = 2 * k + h
                p = h

                @pl.when(s + 1 < seq)
                def _():
                    wait_idx(1 - p)
                    fire_gather(1 - p)

                @pl.when(s + 2 < seq)
                def _():
                    fire_idx(s + 2, p)

                wait_gather(p)

                @pl.when(s >= 2)
                def _():
                    wait_out(p)

                extract(p)
                fire_out(s, p)
            return carry

        lax.fori_loop(0, seq // 2, body, 0)
        wait_out(0)
        wait_out(1)

    return relayout_kernel, gather_kernel


def kernel(indices, table):
    batch, seq = indices.shape
    vocab, dim = table.shape
    relayout_kernel, gather_kernel = _make_kernels(seq, batch, vocab, dim)
    idx_t = jnp.transpose(indices.astype(jnp.int32))
    table_t = jnp.transpose(table)
    lanes = 128
    tail = vocab % lanes
    if tail:
        tail_rows = table[vocab - tail:].reshape(tail * dim)
    else:
        tail_rows = jnp.zeros((lanes,), jnp.float32)
    scratch = relayout_kernel(table_t, tail_rows)
    scratch = scratch.reshape(-1, lanes)
    out_t = gather_kernel(idx_t, scratch)
    return jnp.transpose(out_t, (2, 0, 1))


# speedup vs baseline: 2.4833x; 1.0047x over previous
import functools

import jax
import jax.numpy as jnp
from jax import lax
from jax.experimental import pallas as pl
from jax.experimental.pallas import tpu as pltpu
from jax.experimental.pallas import tpu_sc as plsc


def _iota16():
    return lax.broadcasted_iota(jnp.int32, (16,), 0)


@functools.lru_cache(maxsize=None)
def _make_kernels(seq: int, batch: int, vocab: int, dim: int):
    info = plsc.get_sparse_core_info()
    num_cores, num_subcores = info.num_cores, info.num_subcores
    num_workers = num_cores * num_subcores
    lanes = 128

    n_full_tcols = vocab // lanes
    tail = vocab - n_full_tcols * lanes
    rows_per_tcol = dim
    scratch_rows = (n_full_tcols + (1 if tail else 0)) * rows_per_tcol

    mesh = plsc.VectorSubcoreMesh(core_axis_name="c", subcore_axis_name="s")

    steps = (n_full_tcols + num_workers - 1) // num_workers
    depth = 4
    rounds = (steps + depth - 1) // depth

    scratch_words = scratch_rows * lanes

    @functools.partial(
        pl.kernel,
        mesh=mesh,
        compiler_params=pltpu.CompilerParams(needs_layout_passes=False),
        out_type=jax.ShapeDtypeStruct((scratch_words,), jnp.float32),
        scratch_types=[
            pltpu.VMEM((dim, lanes), jnp.float32),
            pltpu.VMEM((dim, lanes), jnp.float32),
            pltpu.VMEM((dim, lanes), jnp.float32),
            pltpu.VMEM((dim, lanes), jnp.float32),
            pltpu.VMEM((dim * lanes,), jnp.float32),
            pltpu.VMEM((dim * lanes,), jnp.float32),
            pltpu.VMEM((dim * lanes,), jnp.float32),
            pltpu.VMEM((dim * lanes,), jnp.float32),
            pltpu.SemaphoreType.DMA,
            pltpu.SemaphoreType.DMA,
            pltpu.SemaphoreType.DMA,
            pltpu.SemaphoreType.DMA,
            pltpu.SemaphoreType.DMA,
            pltpu.SemaphoreType.DMA,
            pltpu.SemaphoreType.DMA,
            pltpu.SemaphoreType.DMA,
        ],
    )
    def relayout_kernel(tab_hbm, tail_hbm, scr_hbm,
                        vin0, vin1, vin2, vin3, vout0, vout1, vout2, vout3,
                        isem0, isem1, isem2, isem3,
                        osem0, osem1, osem2, osem3):
        wid = lax.axis_index("s") * num_cores + lax.axis_index("c")
        vins = (vin0, vin1, vin2, vin3)
        vouts = (vout0, vout1, vout2, vout3)
        isems = (isem0, isem1, isem2, isem3)
        osems = (osem0, osem1, osem2, osem3)
        iota = _iota16()
        blk_words = dim * lanes

        def col_of(k2):
            return wid + k2 * num_workers

        def fire_in(k2, p):
            c = col_of(k2)

            @pl.when(c < n_full_tcols)
            def _():
                off = pl.multiple_of(c * lanes, lanes)
                pltpu.async_copy(tab_hbm.at[:, pl.ds(off, lanes)],
                                 vins[p], isems[p])

        def transpose_tile(p):
            vin, vout = vins[p], vouts[p]
            rvecs = [iota + 16 * dblk for dblk in range(dim // 16)]

            def tbody(t, carry):
                pt = lax.bitwise_and(iota + t, 15)
                pt32i = lax.shift_left(pt, 5) + iota
                for dblk in range(dim // 16):
                    for bblk in range(lanes // 16):
                        b0 = 16 * bblk
                        src = plsc.load_gather(vin, [rvecs[dblk], pt + b0])
                        plsc.store_scatter(
                            vout, [pt32i + (b0 * 32 + 16 * dblk)], src)
                return carry

            lax.fori_loop(0, 16, tbody, 0)

        def fire_out(k2, p):
            c = col_of(k2)

            @pl.when(c < n_full_tcols)
            def _():
                off = pl.multiple_of(c * blk_words, 8)
                pltpu.async_copy(vouts[p],
                                 scr_hbm.at[pl.ds(off, blk_words)],
                                 osems[p])

        def wait_in(k2, p):
            c = col_of(k2)

            @pl.when(c < n_full_tcols)
            def _():
                pltpu.make_async_copy(tab_hbm.at[:, pl.ds(0, lanes)],
                                      vins[p], isems[p]).wait()

        def wait_out(k2, p):
            c = col_of(k2)

            @pl.when(c < n_full_tcols)
            def _():
                pltpu.make_async_copy(vouts[p],
                                      scr_hbm.at[pl.ds(0, blk_words)],
                                      osems[p]).wait()

        for h in range(depth - 1):
            fire_in(h, h)

        def body(k, carry):
            for h in range(depth):
                k2 = depth * k + h
                p = h
                fire_in(k2 + depth - 1, (h + depth - 1) % depth)
                wait_in(k2, p)

                @pl.when(k2 >= depth)
                def _():
                    wait_out(k2 - depth, p)

                @pl.when(col_of(k2) < n_full_tcols)
                def _():
                    transpose_tile(p)

                fire_out(k2, p)
            return carry

        lax.fori_loop(0, rounds, body, 0)
        for h in range(depth):
            wait_out(depth * rounds - depth + h, h)

        if tail:
            tail_words = tail * dim

            @pl.when(wid == num_workers - 1)
            def _():
                pltpu.sync_copy(tail_hbm, vout0.at[pl.ds(0, tail_words)])
                pltpu.sync_copy(
                    vout0.at[pl.ds(0, tail_words)],
                    scr_hbm.at[pl.ds(n_full_tcols * blk_words, tail_words)])

    @functools.partial(
        pl.kernel,
        mesh=mesh,
        compiler_params=pltpu.CompilerParams(needs_layout_passes=False),
        out_type=jax.ShapeDtypeStruct((seq, dim, batch), jnp.float32),
        scratch_types=[
            pltpu.VMEM((lanes,), jnp.int32),
            pltpu.VMEM((lanes,), jnp.int32),
            pltpu.VMEM((lanes,), jnp.int32),
            pltpu.VMEM((lanes,), jnp.int32),
            pltpu.VMEM((lanes,), jnp.int32),
            pltpu.VMEM((lanes,), jnp.int32),
            pltpu.VMEM((lanes, lanes), jnp.float32),
            pltpu.VMEM((lanes, lanes), jnp.float32),
            pltpu.VMEM((dim, lanes), jnp.float32),
            pltpu.VMEM((dim, lanes), jnp.float32),
            pltpu.SemaphoreType.DMA,
            pltpu.SemaphoreType.DMA,
            pltpu.SemaphoreType.DMA,
            pltpu.SemaphoreType.DMA,
            pltpu.SemaphoreType.DMA,
            pltpu.SemaphoreType.DMA,
        ],
    )
    def gather_kernel(idx_hbm, scr_hbm, out_hbm,
                      vidx0, vidx1, jref0, jref1, rbref0, rbref1,
                      g0, g1, vo0, vo1,
                      xsem0, xsem1, gsem0, gsem1, osem0, osem1):
        wid = lax.axis_index("s") * num_cores + lax.axis_index("c")
        boff = pl.multiple_of(wid * lanes, lanes)
        vidxs = (vidx0, vidx1)
        jrefs = (jref0, jref1)
        rbrefs = (rbref0, rbref1)
        gs = (g0, g1)
        vos = (vo0, vo1)
        xsems = (xsem0, xsem1)
        gsems = (gsem0, gsem1)
        osems = (osem0, osem1)
        iota = _iota16()

        def fire_idx(s, p):
            pltpu.async_copy(idx_hbm.at[s, pl.ds(boff, lanes)],
                             vidxs[p], xsems[p])

        def wait_idx(p):
            pltpu.make_async_copy(idx_hbm.at[0, pl.ds(boff, lanes)],
                                  vidxs[p], xsems[p]).wait()

        def fire_gather(p):
            for g in range(lanes // 16):
                iv = vidxs[p][pl.ds(16 * g, 16)]
                jrefs[p][pl.ds(16 * g, 16)] = lax.shift_right_logical(iv, 2)
                rbrefs[p][pl.ds(16 * g, 16)] = lax.shift_left(
                    lax.bitwise_and(iv, 3), 5)
            pltpu.async_copy(scr_hbm.at[jrefs[p]], gs[p], gsems[p])

        def wait_gather(p):
            pltpu.make_async_copy(scr_hbm.at[jrefs[p]], gs[p], gsems[p]).wait()

        def extract(p):
            G, vout = gs[p], vos[p]
            colbases = [rbrefs[p][pl.ds(16 * g, 16)]
                        for g in range(lanes // 16)]
            rows = [iota + 16 * g for g in range(lanes // 16)]

            def tbody(t, carry):
                pt = lax.bitwise_and(iota + t, dim - 1)
                for g in range(lanes // 16):
                    val = plsc.load_gather(G, [rows[g], colbases[g] + pt])
                    plsc.store_scatter(vout, [pt, rows[g]], val)
                return carry

            lax.fori_loop(0, dim, tbody, 0)

        def fire_out(s, p):
            pltpu.async_copy(vos[p], out_hbm.at[s].at[:, pl.ds(boff, lanes)],
                             osems[p])

        def wait_out(p):
            pltpu.make_async_copy(vos[p],
                                  out_hbm.at[0].at[:, pl.ds(boff, lanes)],
                                  osems[p]).wait()

        fire_idx(0, 0)
        wait_idx(0)
        fire_gather(0)
        fire_idx(1, 1)

        def body(k, carry):
            for h in range(2):
                s = 2 * k + h
                p = h

                @pl.when(s + 1 < seq)
                def _():
                    wait_idx(1 - p)
                    fire_gather(1 - p)

                @pl.when(s + 2 < seq)
                def _():
                    fire_idx(s + 2, p)

                wait_gather(p)

                @pl.when(s >= 2)
                def _():
                    wait_out(p)

                extract(p)
                fire_out(s, p)
            return carry

        lax.fori_loop(0, seq // 2, body, 0)
        wait_out(0)
        wait_out(1)

    return relayout_kernel, gather_kernel


def kernel(indices, table):
    batch, seq = indices.shape
    vocab, dim = table.shape
    relayout_kernel, gather_kernel = _make_kernels(seq, batch, vocab, dim)
    idx_t = jnp.transpose(indices.astype(jnp.int32))
    table_t = jnp.transpose(table)
    lanes = 128
    tail = vocab % lanes
    if tail:
        tail_rows = table[vocab - tail:].reshape(tail * dim)
    else:
        tail_rows = jnp.zeros((lanes,), jnp.float32)
    scratch = relayout_kernel(table_t, tail_rows)
    scratch = scratch.reshape(-1, lanes)
    out_t = gather_kernel(idx_t, scratch)
    return jnp.transpose(out_t, (2, 0, 1))
